# baseline (speedup 1.0000x reference)
import functools

import jax
import jax.numpy as jnp
from jax import lax
from jax.experimental import pallas as pl
from jax.experimental.pallas import tpu as pltpu
from jax.experimental.pallas import tpu_sc as plsc

B = 16384
D = 256
NC, NS = 2, 16
NW = NC * NS
IDX_MINOR = 64
NCH = 1
BC = B // NCH
ROWS_PER_W = BC // NW
CHUNKS = ROWS_PER_W // IDX_MINOR
NBUF = min(6, CHUNKS)


def _sc_gather(idx2, table):
    mesh = plsc.VectorSubcoreMesh(
        core_axis_name="c", subcore_axis_name="s",
        num_cores=NC, num_subcores=NS)

    @functools.partial(
        pl.kernel,
        mesh=mesh,
        out_type=jax.ShapeDtypeStruct((BC, D), jnp.float32),
        scratch_types=(
            [pltpu.VMEM((CHUNKS, IDX_MINOR), jnp.int32)]
            + [pltpu.VMEM((IDX_MINOR, D), jnp.float32) for _ in range(NBUF)]
            + [pltpu.SemaphoreType.DMA for _ in range(2 * NBUF)]
        ),
    )
    def gather_k(idx_hbm, table_hbm, out_hbm, idx_v, *scratch):
        bufs = scratch[:NBUF]
        gsems = scratch[NBUF:2 * NBUF]
        osems = scratch[2 * NBUF:]
        wid = lax.axis_index("s") * NC + lax.axis_index("c")
        pltpu.sync_copy(idx_hbm.at[pl.ds(wid * CHUNKS, CHUNKS)], idx_v)
        gathers = [None] * CHUNKS
        outs = [None] * CHUNKS

        def fire_gather(j):
            k = j % NBUF
            gathers[j] = pltpu.async_copy(
                table_hbm.at[idx_v.at[j]], bufs[k], gsems[k])

        def fire_out(j):
            k = j % NBUF
            dst = out_hbm.at[pl.ds(wid * ROWS_PER_W + j * IDX_MINOR,
                                   IDX_MINOR)]
            outs[j] = pltpu.async_copy(bufs[k], dst, osems[k])

        for j in range(min(NBUF, CHUNKS)):
            fire_gather(j)
        for j in range(CHUNKS):
            gathers[j].wait()
            fire_out(j)
            nxt = j + NBUF
            if nxt < CHUNKS:
                outs[nxt - NBUF].wait()
                fire_gather(nxt)
        for j in range(max(0, CHUNKS - NBUF), CHUNKS):
            outs[j].wait()

    return gather_k(idx2, table)


def _dense_body(p_ref, v4_ref, w1_ref, wct_ref, w6_ref, b6_ref, y_ref):
    f32 = jnp.float32
    pb = p_ref[...]
    v1t = jnp.tanh(jnp.dot(w1_ref[...], pb, preferred_element_type=f32))
    rows = lax.broadcasted_iota(jnp.int32, (8, 1), 0).astype(f32)
    oh3t = (pb[3:4, :] == rows).astype(f32)
    oh4t = (pb[4:5, :] == rows).astype(f32)
    ones = pb[5:6, :]
    ut = jnp.concatenate([v1t, oh3t, oh4t[0:7, :], ones], axis=0)
    v5t = jnp.tanh(jnp.dot(wct_ref[...], ut, preferred_element_type=f32))
    y = (jnp.dot(v4_ref[...], w6_ref[0:D, :], preferred_element_type=f32)
         + lax.dot_general(v5t, w6_ref[D:2 * D, :],
                           (((0,), (0,)), ((), ())),
                           preferred_element_type=f32)
         + b6_ref[...])
    y_ref[...] = jnp.tanh(y)


def kernel(p1, p2, p5, p3, p4, p6, W1, b1, E2, E3, E4, W5, b5, W6, b6):
    f32 = jnp.float32
    W1e = jnp.zeros((8, 8), f32)
    W1e = W1e.at[:, 0:3].set(W1.T)
    W1e = W1e.at[:, 5].set(b1)
    WcT = jnp.zeros((D, 24), f32)
    WcT = WcT.at[:, 0:8].set(W5[0:8, :].T)
    hi = jax.lax.Precision.HIGHEST
    WcT = WcT.at[:, 8:8 + E2.shape[0]].set(
        jnp.dot(E2, W5[8:16, :], precision=hi).T)
    WcT = WcT.at[:, 16:16 + E3.shape[0]].set(
        jnp.dot(E3, W5[16:24, :], precision=hi).T)
    WcT = WcT.at[:, 23].set(b5)

    P = jnp.concatenate([
        p1.T, p2.T, p5.T,
        p3.astype(f32)[None, :], p4.astype(f32)[None, :],
        jnp.ones((1, B), f32), jnp.zeros((2, B), f32)], axis=0)

    idx2 = p6.astype(jnp.int32).reshape(B // IDX_MINOR, IDX_MINOR)
    rows_per_chunk = BC // IDX_MINOR
    v4s = [_sc_gather(idx2[c * rows_per_chunk:(c + 1) * rows_per_chunk], E4)
           for c in range(NCH)]

    BK = 4096
    blocks_per_chunk = BC // BK
    rep = lambda i: (0, 0)
    y = None
    for c in range(NCH):
        off = c * blocks_per_chunk
        in_specs = [
            pl.BlockSpec((8, BK), lambda i, off=off: (0, i + off)),
            pl.BlockSpec((BK, D), lambda i: (i, 0)),
            pl.BlockSpec((8, 8), rep),
            pl.BlockSpec((D, 24), rep),
            pl.BlockSpec((2 * D, 128), rep),
            pl.BlockSpec((1, 128), rep),
        ]
        args = [P, v4s[c], W1e, WcT, W6, b6[None, :]]
        aliases = {}
        body = _dense_body
        if c > 0:
            in_specs.append(pl.BlockSpec(memory_space=pl.ANY))
            args.append(y)
            aliases = {len(args) - 1: 0}
            body = lambda *refs: _dense_body(*refs[:-2], refs[-1])
        y = pl.pallas_call(
            body,
            grid=(blocks_per_chunk,),
            in_specs=in_specs,
            out_specs=pl.BlockSpec((BK, 128), lambda i, off=off: (i + off, 0)),
            out_shape=jax.ShapeDtypeStruct((B, 128), f32),
            input_output_aliases=aliases,
        )(*args)
    return y


# speedup vs baseline: 1.0071x; 1.0071x over previous
import functools

import jax
import jax.numpy as jnp
from jax import lax
from jax.experimental import pallas as pl
from jax.experimental.pallas import tpu as pltpu
from jax.experimental.pallas import tpu_sc as plsc

B = 16384
D = 256
NC, NS = 2, 16
NW = NC * NS
IDX_MINOR = 64
ROWS_PER_W = B // NW
CHUNKS = ROWS_PER_W // IDX_MINOR
NBUF = min(6, CHUNKS)
BK = 4096


def _sc_gather(idx2, table):
    mesh = plsc.VectorSubcoreMesh(
        core_axis_name="c", subcore_axis_name="s",
        num_cores=NC, num_subcores=NS)

    @functools.partial(
        pl.kernel,
        mesh=mesh,
        out_type=jax.ShapeDtypeStruct((B, D), jnp.float32),
        scratch_types=(
            [pltpu.VMEM((CHUNKS, IDX_MINOR), jnp.int32)]
            + [pltpu.VMEM((IDX_MINOR, D), jnp.float32) for _ in range(NBUF)]
            + [pltpu.SemaphoreType.DMA for _ in range(2 * NBUF)]
        ),
    )
    def gather_k(idx_hbm, table_hbm, out_hbm, idx_v, *scratch):
        bufs = scratch[:NBUF]
        gsems = scratch[NBUF:2 * NBUF]
        osems = scratch[2 * NBUF:]
        wid = lax.axis_index("s") * NC + lax.axis_index("c")
        pltpu.sync_copy(idx_hbm.at[pl.ds(wid * CHUNKS, CHUNKS)], idx_v)
        gathers = [None] * CHUNKS
        outs = [None] * CHUNKS

        def fire_gather(j):
            k = j % NBUF
            gathers[j] = pltpu.async_copy(
                table_hbm.at[idx_v.at[j]], bufs[k], gsems[k])

        def fire_out(j):
            k = j % NBUF
            dst = out_hbm.at[pl.ds(wid * ROWS_PER_W + j * IDX_MINOR,
                                   IDX_MINOR)]
            outs[j] = pltpu.async_copy(bufs[k], dst, osems[k])

        for j in range(min(NBUF, CHUNKS)):
            fire_gather(j)
        for j in range(CHUNKS):
            gathers[j].wait()
            fire_out(j)
            nxt = j + NBUF
            if nxt < CHUNKS:
                outs[nxt - NBUF].wait()
                fire_gather(nxt)
        for j in range(max(0, CHUNKS - NBUF), CHUNKS):
            outs[j].wait()

    return gather_k(idx2, table)


def _dense_body(p_ref, v4_ref, w1_ref, wct_ref, w6_ref, b6_ref, y_ref):
    f32 = jnp.float32
    pb = p_ref[...]
    v1t = jnp.tanh(jnp.dot(w1_ref[...], pb, preferred_element_type=f32))
    rows = lax.broadcasted_iota(jnp.int32, (8, 1), 0).astype(f32)
    oh3t = (pb[3:4, :] == rows).astype(f32)
    oh4t = (pb[4:5, :] == rows).astype(f32)
    ones = pb[5:6, :]
    ut = jnp.concatenate([v1t, oh3t, oh4t[0:7, :], ones], axis=0)
    v5t = jnp.tanh(jnp.dot(wct_ref[...], ut, preferred_element_type=f32))
    y = (jnp.dot(v4_ref[...], w6_ref[0:D, :], preferred_element_type=f32)
         + lax.dot_general(v5t, w6_ref[D:2 * D, :],
                           (((0,), (0,)), ((), ())),
                           preferred_element_type=f32)
         + b6_ref[...])
    y_ref[...] = jnp.tanh(y)


def kernel(p1, p2, p5, p3, p4, p6, W1, b1, E2, E3, E4, W5, b5, W6, b6):
    f32 = jnp.float32
    W1e = jnp.zeros((8, 8), f32)
    W1e = W1e.at[:, 0:3].set(W1.T)
    W1e = W1e.at[:, 5].set(b1)
    hi = jax.lax.Precision.HIGHEST
    WcT = jnp.zeros((D, 24), f32)
    WcT = WcT.at[:, 0:8].set(W5[0:8, :].T)
    WcT = WcT.at[:, 8:8 + E2.shape[0]].set(
        jnp.dot(E2, W5[8:16, :], precision=hi).T)
    WcT = WcT.at[:, 16:16 + E3.shape[0]].set(
        jnp.dot(E3, W5[16:24, :], precision=hi).T)
    WcT = WcT.at[:, 23].set(b5)

    P = jnp.concatenate([
        p1.T, p2.T, p5.T,
        p3.astype(f32)[None, :], p4.astype(f32)[None, :],
        jnp.ones((1, B), f32), jnp.zeros((2, B), f32)], axis=0)

    idx2 = p6.astype(jnp.int32).reshape(B // IDX_MINOR, IDX_MINOR)
    v4 = _sc_gather(idx2, E4)

    rep = lambda i: (0, 0)
    y = pl.pallas_call(
        _dense_body,
        grid=(B // BK,),
        in_specs=[
            pl.BlockSpec((8, BK), lambda i: (0, i)),
            pl.BlockSpec((BK, D), lambda i: (i, 0)),
            pl.BlockSpec((8, 8), rep),
            pl.BlockSpec((D, 24), rep),
            pl.BlockSpec((2 * D, 128), rep),
            pl.BlockSpec((1, 128), rep),
        ],
        out_specs=pl.BlockSpec((BK, 128), lambda i: (i, 0)),
        out_shape=jax.ShapeDtypeStruct((B, 128), f32),
    )(P, v4, W1e, WcT, W6, b6[None, :])
    return y
